# baseline (speedup 1.0000x reference)
import jax
import jax.numpy as jnp
from jax import lax
from jax.experimental import pallas as pl
from jax.experimental.pallas import tpu as pltpu
from jax.experimental.pallas import tpu_sc as plsc

_NB = 4
_NC = 768
_D = 8192
_K = 16
_QD = 2048
_NQ = _D // _QD
_NCG = _NC // _K
_TASKS = _NB * _NCG * _NQ
_NW = 32
_TPW = _TASKS // _NW
_NSLOT = 3


def _body(in_hbm, out_hbm,
          idx0, idx1, idx2, buf0, buf1, buf2, sem_g, sem_s):
    cid = lax.axis_index("c")
    sid = lax.axis_index("s")
    wid = cid * 16 + sid
    t0 = wid * _TPW

    iv = lax.iota(jnp.int32, _K)
    idxs = (idx0, idx1, idx2)
    bufs = (buf0, buf1, buf2)

    def task(k):
        t = t0 + k
        b = t // (_NCG * _NQ)
        rem = t % (_NCG * _NQ)
        cg = rem // _NQ
        q = rem % _NQ
        return b, cg * _K, q * _QD

    def start_gather(k, slot):
        b, o0, qq = task(k)
        idxs[slot][...] = (_NC - 1 - o0) - iv
        src = in_hbm.at[b, :, pl.ds(qq, _QD)]
        return pltpu.async_copy(src.at[idxs[slot]], bufs[slot], sem_g)

    def start_store(k, slot):
        b, o0, qq = task(k)
        dst = out_hbm.at[b, pl.ds(o0, _K), pl.ds(qq, _QD)]
        return pltpu.async_copy(bufs[slot], dst, sem_s)

    gath = {}
    for k in range(_NSLOT):
        gath[k] = start_gather(k, k % _NSLOT)
    st = {}
    for k in range(_TPW):
        if k > 0:
            st[k - 1].wait()
            if k - 1 + _NSLOT < _TPW:
                gath[k - 1 + _NSLOT] = start_gather(
                    k - 1 + _NSLOT, (k - 1) % _NSLOT)
        gath[k].wait()
        st[k] = start_store(k, k % _NSLOT)
    st[_TPW - 1].wait()


@jax.jit
def _permute(x):
    mesh = plsc.VectorSubcoreMesh(core_axis_name="c", subcore_axis_name="s")
    return pl.kernel(
        _body,
        mesh=mesh,
        out_type=jax.ShapeDtypeStruct((_NB, _NC, _D), jnp.float32),
        scratch_types=[
            pltpu.VMEM((_K,), jnp.int32),
            pltpu.VMEM((_K,), jnp.int32),
            pltpu.VMEM((_K,), jnp.int32),
            pltpu.VMEM((_K, _QD), jnp.float32),
            pltpu.VMEM((_K, _QD), jnp.float32),
            pltpu.VMEM((_K, _QD), jnp.float32),
            pltpu.SemaphoreType.DMA,
            pltpu.SemaphoreType.DMA,
        ],
    )(x)


def kernel(input):
    return _permute(input)


# speedup vs baseline: 1.0216x; 1.0216x over previous
import jax
import jax.numpy as jnp
from jax import lax
from jax.experimental import pallas as pl
from jax.experimental.pallas import tpu as pltpu
from jax.experimental.pallas import tpu_sc as plsc

_NB = 4
_NC = 768
_D = 8192
_K = 16
_QD = 2048
_NQ = _D // _QD
_NCG = _NC // _K
_TASKS = _NB * _NCG * _NQ
_NW = 32
_TPW = _TASKS // _NW
_NCGW = _TPW // _NQ
_NSLOT = 3


def _body(in_hbm, out_hbm,
          idx0, idx1, idx2, idx3, idx4, idx5,
          buf0, buf1, buf2, sem_g, sem_s):
    cid = lax.axis_index("c")
    sid = lax.axis_index("s")
    wid = cid * 16 + sid
    t0 = wid * _TPW

    iv = lax.iota(jnp.int32, _K)
    idxs = (idx0, idx1, idx2, idx3, idx4, idx5)
    bufs = (buf0, buf1, buf2)

    def task(k):
        t = t0 + k
        b = t // (_NCG * _NQ)
        rem = t % (_NCG * _NQ)
        cg = rem // _NQ
        q = rem % _NQ
        return b, cg * _K, q * _QD

    for j in range(_NCGW):
        _, o0, _ = task(4 * j)
        idxs[j][...] = (_NC - 1 - o0) - iv

    def start_gather(k, slot):
        b, _, qq = task(k)
        src = in_hbm.at[b, :, pl.ds(qq, _QD)]
        return pltpu.async_copy(src.at[idxs[k // _NQ]], bufs[slot], sem_g)

    def start_store(k, slot):
        b, o0, qq = task(k)
        dst = out_hbm.at[b, pl.ds(o0, _K), pl.ds(qq, _QD)]
        return pltpu.async_copy(bufs[slot], dst, sem_s)

    gath = {}
    for k in range(_NSLOT):
        gath[k] = start_gather(k, k % _NSLOT)
    st = {}
    for k in range(_TPW):
        if k > 0:
            st[k - 1].wait()
            if k - 1 + _NSLOT < _TPW:
                gath[k - 1 + _NSLOT] = start_gather(
                    k - 1 + _NSLOT, (k - 1) % _NSLOT)
        gath[k].wait()
        st[k] = start_store(k, k % _NSLOT)
    st[_TPW - 1].wait()


@jax.jit
def _permute(x):
    mesh = plsc.VectorSubcoreMesh(core_axis_name="c", subcore_axis_name="s")
    return pl.kernel(
        _body,
        mesh=mesh,
        out_type=jax.ShapeDtypeStruct((_NB, _NC, _D), jnp.float32),
        scratch_types=[
            pltpu.VMEM((_K,), jnp.int32),
            pltpu.VMEM((_K,), jnp.int32),
            pltpu.VMEM((_K,), jnp.int32),
            pltpu.VMEM((_K,), jnp.int32),
            pltpu.VMEM((_K,), jnp.int32),
            pltpu.VMEM((_K,), jnp.int32),
            pltpu.VMEM((_K, _QD), jnp.float32),
            pltpu.VMEM((_K, _QD), jnp.float32),
            pltpu.VMEM((_K, _QD), jnp.float32),
            pltpu.SemaphoreType.DMA,
            pltpu.SemaphoreType.DMA,
        ],
    )(x)


def kernel(input):
    return _permute(input)
